# Initial kernel scaffold; baseline (speedup 1.0000x reference)
import jax
import jax.numpy as jnp
from jax.experimental import pallas as pl


def kernel(model_name, pretrained_dataset, model_type, model_owner, model_architecture, model_task, numeric_features, T_name, T_ds, T_type, T_owner, T_arch, T_task, W1, b1, W2, b2):
    raise NotImplementedError("write your pallas kernel here")


# speedup vs baseline: 1.0889x; 1.0889x over previous
import functools

import jax
import jax.numpy as jnp
from jax import lax
from jax.experimental import pallas as pl
from jax.experimental.pallas import tpu as pltpu
from jax.experimental.pallas import tpu_sc as plsc

B = 16384
ED = 16
NC = 2
NS = 16
NW = NC * NS
BPW = B // NW
CHUNK = 128
NCHUNK = BPW // CHUNK


def _gather6(idx_packed, t_name, t_ds, t_type, t_owner, t_arch, t_task):
    mesh = plsc.VectorSubcoreMesh(core_axis_name="c", subcore_axis_name="s")

    @functools.partial(
        pl.kernel,
        out_type=[jax.ShapeDtypeStruct((B, ED), jnp.float32) for _ in range(6)],
        mesh=mesh,
        scratch_types=[
            pltpu.VMEM((6, NCHUNK, CHUNK), jnp.int32),
            pltpu.VMEM((6, BPW, ED), jnp.float32),
            pltpu.SemaphoreType.DMA,
            pltpu.SemaphoreType.DMA,
        ],
        compiler_params=pltpu.CompilerParams(use_tc_tiling_on_sc=False),
    )
    def k(idx_hbm, tn, td, tt, to, ta, tk,
          o0, o1, o2, o3, o4, o5, idx_v, rows_v, gsem, wsem):
        wid = lax.axis_index("s") * NC + lax.axis_index("c")
        base = wid * BPW
        tables = [tn, td, tt, to, ta, tk]
        outs = [o0, o1, o2, o3, o4, o5]
        pltpu.sync_copy(idx_hbm.at[wid], idx_v)
        copies = []
        for t in range(6):
            for c in range(NCHUNK):
                cp = pltpu.async_copy(
                    tables[t].at[idx_v.at[t, c]],
                    rows_v.at[t, pl.ds(c * CHUNK, CHUNK)],
                    gsem,
                )
                copies.append(cp)
        for cp in copies:
            cp.wait()
        writes = []
        for t in range(6):
            writes.append(
                pltpu.async_copy(rows_v.at[t], outs[t].at[pl.ds(base, BPW)], wsem)
            )
        for cp in writes:
            cp.wait()

    return k(idx_packed, t_name, t_ds, t_type, t_owner, t_arch, t_task)


def _mlp_body(e0, e1, e2, e3, e4, e5, nf, w1, b1, w2, b2, out):
    num = jnp.dot(nf[:], w1[:], preferred_element_type=jnp.float32) + b1[:]
    feats = jnp.concatenate([e0[:], e1[:], e2[:], e3[:], e4[:], e5[:], num], axis=-1)
    acc = jnp.dot(feats, w2[:], preferred_element_type=jnp.float32) + b2[:]
    out[:] = jnp.maximum(acc, 0.0)


def _mlp(e_list, nf, w1, b1, w2, b2):
    BB = 2048
    grid = (B // BB,)
    espec = pl.BlockSpec((BB, ED), lambda i: (i, 0))
    return pl.pallas_call(
        _mlp_body,
        grid=grid,
        in_specs=[espec] * 6 + [
            pl.BlockSpec((BB, 6), lambda i: (i, 0)),
            pl.BlockSpec((6, 20), lambda i: (0, 0)),
            pl.BlockSpec((1, 20), lambda i: (0, 0)),
            pl.BlockSpec((116, 64), lambda i: (0, 0)),
            pl.BlockSpec((1, 64), lambda i: (0, 0)),
        ],
        out_specs=pl.BlockSpec((BB, 64), lambda i: (i, 0)),
        out_shape=jax.ShapeDtypeStruct((B, 64), jnp.float32),
        compiler_params=pltpu.CompilerParams(
            dimension_semantics=("parallel",),
        ),
    )(*e_list, nf, w1, b1, w2, b2)


def kernel(model_name, pretrained_dataset, model_type, model_owner,
           model_architecture, model_task, numeric_features,
           T_name, T_ds, T_type, T_owner, T_arch, T_task, W1, b1, W2, b2):
    idx = jnp.stack([
        model_name.astype(jnp.int32),
        pretrained_dataset.astype(jnp.int32),
        model_type.astype(jnp.int32),
        model_owner.astype(jnp.int32),
        model_architecture.astype(jnp.int32),
        model_task.astype(jnp.int32),
    ], axis=0)
    idx = idx.reshape(6, NW, NCHUNK, CHUNK).transpose(1, 0, 2, 3)
    e = _gather6(idx, T_name, T_ds, T_type, T_owner, T_arch, T_task)
    return _mlp(e, numeric_features,
                W1, b1.reshape(1, 20), W2, b2.reshape(1, 64))
